# baseline (speedup 1.0000x reference)
import functools

import jax
import jax.numpy as jnp
from jax import lax
from jax.experimental import pallas as pl
from jax.experimental.pallas import tpu as pltpu
from jax.experimental.pallas import tpu_sc as plsc

EPS = 1e-12

_NUM_CORES = 2
_NUM_SUBCORES = 16
_NW = _NUM_CORES * _NUM_SUBCORES
_CHUNK = 32
_NBUF = 4


@functools.lru_cache(maxsize=None)
def _make_sc_gather(n_tokens: int, hidden: int):
    assert n_tokens % (_NW * _CHUNK) == 0
    per_w = n_tokens // _NW
    n_chunks = per_w // _CHUNK
    mesh = plsc.VectorSubcoreMesh(core_axis_name="c", subcore_axis_name="s")

    @functools.partial(
        pl.kernel,
        out_type=jax.ShapeDtypeStruct((n_tokens, hidden), jnp.float32),
        mesh=mesh,
        scratch_types=(
            [pltpu.VMEM((per_w,), jnp.int32)]
            + [pltpu.VMEM((_CHUNK, hidden), jnp.float32)] * _NBUF
            + [pltpu.SemaphoreType.DMA] * _NBUF
            + [pltpu.SemaphoreType.DMA] * _NBUF
        ),
    )
    def gather(word_hbm, ids_hbm, out_hbm, idx_v, *bufs_sems):
        bufs = bufs_sems[:_NBUF]
        gsems = bufs_sems[_NBUF:2 * _NBUF]
        ssems = bufs_sems[2 * _NBUF:]
        c = lax.axis_index("c")
        s = lax.axis_index("s")
        wid = s * _NUM_CORES + c
        base = wid * per_w
        pltpu.sync_copy(ids_hbm.at[pl.ds(base, per_w)], idx_v)
        gathers = [None] * n_chunks
        stores = [None] * n_chunks
        store_waited = [False] * n_chunks
        depth = min(_NBUF - 1, n_chunks)
        for j in range(depth):
            gathers[j] = pltpu.async_copy(
                word_hbm.at[idx_v.at[pl.ds(j * _CHUNK, _CHUNK)]],
                bufs[j % _NBUF], gsems[j % _NBUF])
        for j in range(n_chunks):
            gathers[j].wait()
            stores[j] = pltpu.async_copy(
                bufs[j % _NBUF], out_hbm.at[pl.ds(base + j * _CHUNK, _CHUNK)],
                ssems[j % _NBUF])
            nxt = j + depth
            if nxt < n_chunks:
                prev = nxt - _NBUF
                if prev >= 0:
                    stores[prev].wait()
                    store_waited[prev] = True
                gathers[nxt] = pltpu.async_copy(
                    word_hbm.at[idx_v.at[pl.ds(nxt * _CHUNK, _CHUNK)]],
                    bufs[nxt % _NBUF], gsems[nxt % _NBUF])
        for j in range(n_chunks):
            if not store_waited[j]:
                stores[j].wait()

    return gather


def _ln_body(emb_ref, pos_ref, pos2_ref, w_ref, b_ref, out_ref):
    x = pos_ref[...] * emb_ref[...] + pos2_ref[...]
    mean = jnp.mean(x, axis=-1, keepdims=True)
    xc = x - mean
    var = jnp.mean(xc * xc, axis=-1, keepdims=True)
    y = xc * lax.rsqrt(var + EPS)
    out_ref[...] = y * w_ref[...] + b_ref[...]


def _ln_call(emb, pos, pos2, w, b, block_tokens: int):
    n, hidden = emb.shape
    s_len = pos.shape[0]
    assert n % block_tokens == 0 and s_len % block_tokens == 0
    s_blocks = s_len // block_tokens
    batch = n // s_len
    return pl.pallas_call(
        _ln_body,
        grid=(s_blocks, batch),
        in_specs=[
            pl.BlockSpec((block_tokens, hidden), lambda j, bi: (bi * s_blocks + j, 0)),
            pl.BlockSpec((block_tokens, hidden), lambda j, bi: (j, 0)),
            pl.BlockSpec((block_tokens, hidden), lambda j, bi: (j, 0)),
            pl.BlockSpec((1, hidden), lambda j, bi: (0, 0)),
            pl.BlockSpec((1, hidden), lambda j, bi: (0, 0)),
        ],
        out_specs=pl.BlockSpec(
            (block_tokens, hidden), lambda j, bi: (bi * s_blocks + j, 0)),
        out_shape=jax.ShapeDtypeStruct((n, hidden), jnp.float32),
    )(emb, pos, pos2, w, b)


_FK = 16


@functools.lru_cache(maxsize=None)
def _make_sc_fused(n_tokens: int, hidden: int, s_len: int):
    assert n_tokens % (_NW * _FK) == 0
    per_w = n_tokens // _NW
    n_chunks = per_w // _FK
    n_sl = hidden // 16
    mesh = plsc.VectorSubcoreMesh(core_axis_name="c", subcore_axis_name="s")
    row_t = pltpu.VMEM((_FK, hidden), jnp.float32)

    @functools.partial(
        pl.kernel,
        out_type=(
            jax.ShapeDtypeStruct((n_tokens, hidden), jnp.float32),
            jax.ShapeDtypeStruct((n_tokens, hidden), jnp.float32),
        ),
        mesh=mesh,
        scratch_types=(
            [pltpu.VMEM((per_w,), jnp.int32)]
            + [row_t] * 3
            + [row_t] * 3
            + [row_t] * 2 + [row_t] * 2
            + [pltpu.VMEM((hidden,), jnp.float32)] * 2
            + [pltpu.SemaphoreType.DMA] * 13
        ),
    )
    def fused(word, ids, pos, pos2, w, b, emb_out, ie_out, idx_v,
              r0, r1, r2, e0, e1, e2, p0, p1, q0, q1, wv, bv,
              g0, g1, g2, sp0, sp1, sq0, sq1, sr0, sr1, sr2, se0, se1, se2):
        rows = (r0, r1, r2)
        ebuf = (e0, e1, e2)
        posb = (p0, p1)
        pos2b = (q0, q1)
        gsem = (g0, g1, g2)
        psem = (sp0, sp1)
        qsem = (sq0, sq1)
        rsem = (sr0, sr1, sr2)
        esem = (se0, se1, se2)
        c = lax.axis_index("c")
        s = lax.axis_index("s")
        wid = s * _NUM_CORES + c
        base = wid * per_w
        pos_base = lax.rem(base, s_len)
        pltpu.sync_copy(ids.at[pl.ds(base, per_w)], idx_v)
        pltpu.sync_copy(w, wv)
        pltpu.sync_copy(b, bv)

        def fire(j, b3, b2):
            pltpu.async_copy(
                word.at[idx_v.at[pl.ds(j * _FK, _FK)]], rows[b3], gsem[b3])
            pltpu.async_copy(
                pos.at[pl.ds(pos_base + j * _FK, _FK)], posb[b2], psem[b2])
            pltpu.async_copy(
                pos2.at[pl.ds(pos_base + j * _FK, _FK)], pos2b[b2], qsem[b2])

        def wait_in(b3, b2):
            pltpu.make_async_copy(word.at[pl.ds(0, _FK)], rows[b3], gsem[b3]).wait()
            pltpu.make_async_copy(pos.at[pl.ds(0, _FK)], posb[b2], psem[b2]).wait()
            pltpu.make_async_copy(pos2.at[pl.ds(0, _FK)], pos2b[b2], qsem[b2]).wait()

        def store(j, b3):
            pltpu.async_copy(
                ebuf[b3], emb_out.at[pl.ds(base + j * _FK, _FK)], esem[b3])
            pltpu.async_copy(
                rows[b3], ie_out.at[pl.ds(base + j * _FK, _FK)], rsem[b3])

        def drain_store(b3):
            pltpu.make_async_copy(
                ebuf[b3], emb_out.at[pl.ds(0, _FK)], esem[b3]).wait()
            pltpu.make_async_copy(
                rows[b3], ie_out.at[pl.ds(0, _FK)], rsem[b3]).wait()

        inv_h = 1.0 / hidden

        def _allreduce_sum(x):
            lanes = lax.iota(jnp.int32, 16)
            for sh in (8, 4, 2, 1):
                perm = lax.bitwise_xor(lanes, jnp.full((16,), sh, jnp.int32))
                x = x + x.at[perm].get(mode="promise_in_bounds")
            return x

        def compute(b3, b2):
            rb, eb, pb, qb = rows[b3], ebuf[b3], posb[b2], pos2b[b2]

            def tbody(t, carry):
                sacc = jnp.zeros((16,), jnp.float32)
                qacc = jnp.zeros((16,), jnp.float32)
                for sl in range(n_sl):
                    d = pl.ds(sl * 16, 16)
                    x = pb[t, d] * rb[t, d] + qb[t, d]
                    eb[t, d] = x
                    sacc = sacc + x
                    qacc = qacc + x * x
                mean = _allreduce_sum(sacc) * inv_h
                msq = _allreduce_sum(qacc) * inv_h
                v = msq - mean * mean + EPS
                i = lax.bitcast_convert_type(v, jnp.int32)
                i = jnp.full((16,), 0x5F3759DF, jnp.int32) - (i >> 1)
                y = lax.bitcast_convert_type(i, jnp.float32)
                for _ in range(3):
                    y = y * (1.5 - 0.5 * v * y * y)
                for sl in range(n_sl):
                    d = pl.ds(sl * 16, 16)
                    xx = eb[t, d]
                    eb[t, d] = ((xx - mean) * y) * wv[d] + bv[d]
                return carry

            lax.fori_loop(0, _FK, tbody, 0)

        fire(0, 0, 0)
        fire(1, 1, 1)
        wait_in(0, 0)
        compute(0, 0)
        store(0, 0)
        fire(2, 2, 0)
        wait_in(1, 1)
        compute(1, 1)
        store(1, 1)

        def outer(it, carry):
            g = 2 + 6 * it
            for k in range(6):
                j = g + k
                b3 = (2 + k) % 3
                b2 = (2 + k) % 2
                b3n = (b3 + 1) % 3
                b2n = (b2 + 1) % 2
                drain_store(b3n)

                @pl.when(j < n_chunks - 1)
                def _():
                    fire(j + 1, b3n, b2n)

                wait_in(b3, b2)
                compute(b3, b2)
                store(j, b3)
            return carry

        lax.fori_loop(0, (n_chunks - 2) // 6, outer, 0)
        drain_store((n_chunks - 2) % 3)
        drain_store((n_chunks - 1) % 3)

    return fused


def kernel(input_ids, word_emb, pos_emb, pos_emb2, ln_weight, ln_bias):
    b, s = input_ids.shape
    hidden = word_emb.shape[1]
    n = b * s
    ids = input_ids.reshape(n).astype(jnp.int32)
    embeddings, inputs_embeds = _make_sc_fused(n, hidden, s)(
        word_emb, ids, pos_emb[:s], pos_emb2[:s], ln_weight, ln_bias)
    return (embeddings.reshape(b, s, hidden),
            inputs_embeds.reshape(b, s, hidden))


# speedup vs baseline: 1.6812x; 1.6812x over previous
import functools

import jax
import jax.numpy as jnp
from jax import lax
from jax.experimental import pallas as pl
from jax.experimental.pallas import tpu as pltpu
from jax.experimental.pallas import tpu_sc as plsc

EPS = 1e-12

_NUM_CORES = 2
_NUM_SUBCORES = 16
_NW = _NUM_CORES * _NUM_SUBCORES
_FK = 16


@functools.lru_cache(maxsize=None)
def _make_sc_fused(n_tokens: int, hidden: int, s_len: int):
    assert n_tokens % (_NW * _FK) == 0
    per_w = n_tokens // _NW
    n_chunks = per_w // _FK
    n_sl = hidden // 16
    mesh = plsc.VectorSubcoreMesh(core_axis_name="c", subcore_axis_name="s")
    row_t = pltpu.VMEM((_FK, hidden), jnp.float32)

    @functools.partial(
        pl.kernel,
        out_type=(
            jax.ShapeDtypeStruct((n_tokens, hidden), jnp.float32),
            jax.ShapeDtypeStruct((n_tokens, hidden), jnp.float32),
        ),
        mesh=mesh,
        scratch_types=(
            [pltpu.VMEM((per_w,), jnp.int32)]
            + [row_t] * 3
            + [row_t] * 3
            + [row_t] * 2 + [row_t] * 2
            + [pltpu.VMEM((hidden,), jnp.float32)] * 2
            + [pltpu.SemaphoreType.DMA] * 13
        ),
    )
    def fused(word, ids, pos, pos2, w, b, emb_out, ie_out, idx_v,
              r0, r1, r2, e0, e1, e2, p0, p1, q0, q1, wv, bv,
              g0, g1, g2, sp0, sp1, sq0, sq1, sr0, sr1, sr2, se0, se1, se2):
        rows = (r0, r1, r2)
        ebuf = (e0, e1, e2)
        posb = (p0, p1)
        pos2b = (q0, q1)
        gsem = (g0, g1, g2)
        psem = (sp0, sp1)
        qsem = (sq0, sq1)
        rsem = (sr0, sr1, sr2)
        esem = (se0, se1, se2)
        c = lax.axis_index("c")
        s = lax.axis_index("s")
        wid = s * _NUM_CORES + c
        base = wid * per_w
        pos_base = lax.rem(base, s_len)
        pltpu.sync_copy(ids.at[pl.ds(base, per_w)], idx_v)
        pltpu.sync_copy(w, wv)
        pltpu.sync_copy(b, bv)

        def fire(j, b3, b2):
            pltpu.async_copy(
                word.at[idx_v.at[pl.ds(j * _FK, _FK)]], rows[b3], gsem[b3])
            pltpu.async_copy(
                pos.at[pl.ds(pos_base + j * _FK, _FK)], posb[b2], psem[b2])
            pltpu.async_copy(
                pos2.at[pl.ds(pos_base + j * _FK, _FK)], pos2b[b2], qsem[b2])

        def wait_in(b3, b2):
            pltpu.make_async_copy(word.at[pl.ds(0, _FK)], rows[b3], gsem[b3]).wait()
            pltpu.make_async_copy(pos.at[pl.ds(0, _FK)], posb[b2], psem[b2]).wait()
            pltpu.make_async_copy(pos2.at[pl.ds(0, _FK)], pos2b[b2], qsem[b2]).wait()

        def store(j, b3):
            pltpu.async_copy(
                ebuf[b3], emb_out.at[pl.ds(base + j * _FK, _FK)], esem[b3])
            pltpu.async_copy(
                rows[b3], ie_out.at[pl.ds(base + j * _FK, _FK)], rsem[b3])

        def drain_store(b3):
            pltpu.make_async_copy(
                ebuf[b3], emb_out.at[pl.ds(0, _FK)], esem[b3]).wait()
            pltpu.make_async_copy(
                rows[b3], ie_out.at[pl.ds(0, _FK)], rsem[b3]).wait()

        inv_h = 1.0 / hidden
        zero = jnp.zeros((16,), jnp.float32)
        lanes = lax.iota(jnp.int32, 16)

        def compute(b3, b2):
            rb, eb, pb, qb = rows[b3], ebuf[b3], posb[b2], pos2b[b2]

            def sl_body(sl, carry):
                d = pl.ds(sl * 16, 16)
                ns, nq = [], []
                for t in range(_FK):
                    x = pb[t, d] * rb[t, d] + qb[t, d]
                    eb[t, d] = x
                    ns.append(carry[t] + x)
                    nq.append(carry[_FK + t] + x * x)
                return tuple(ns) + tuple(nq)

            accs = lax.fori_loop(0, n_sl, sl_body, (zero,) * (2 * _FK))

            def _allreduce(x):
                for sh in (8, 4, 2, 1):
                    perm = lax.bitwise_xor(lanes, jnp.full((16,), sh, jnp.int32))
                    x = x + x.at[perm].get(mode="promise_in_bounds")
                return x

            mean_p = zero
            msq_p = zero
            for t in range(_FK):
                oh = jnp.where(lanes == t, inv_h, 0.0).astype(jnp.float32)
                mean_p = mean_p + _allreduce(accs[t]) * oh
                msq_p = msq_p + _allreduce(accs[_FK + t]) * oh
            v = msq_p - mean_p * mean_p + EPS
            i = lax.bitcast_convert_type(v, jnp.int32)
            i = jnp.full((16,), 0x5F3759DF, jnp.int32) - (i >> 1)
            y = lax.bitcast_convert_type(i, jnp.float32)
            for _ in range(3):
                y = y * (1.5 - 0.5 * v * y * y)
            bms = [mean_p.at[jnp.full((16,), t, jnp.int32)]
                   .get(mode="promise_in_bounds") for t in range(_FK)]
            bys = [y.at[jnp.full((16,), t, jnp.int32)]
                   .get(mode="promise_in_bounds") for t in range(_FK)]

            def sl2_body(sl, carry):
                d = pl.ds(sl * 16, 16)
                wd = wv[d]
                bd = bv[d]
                for t in range(_FK):
                    xx = eb[t, d]
                    eb[t, d] = (xx - bms[t]) * bys[t] * wd + bd
                return carry

            lax.fori_loop(0, n_sl, sl2_body, 0)

        fire(0, 0, 0)
        fire(1, 1, 1)
        wait_in(0, 0)
        compute(0, 0)
        store(0, 0)
        fire(2, 2, 0)
        wait_in(1, 1)
        compute(1, 1)
        store(1, 1)

        def outer(it, carry):
            g = 2 + 6 * it
            for k in range(6):
                j = g + k
                b3 = (2 + k) % 3
                b2 = (2 + k) % 2
                b3n = (b3 + 1) % 3
                b2n = (b2 + 1) % 2
                drain_store(b3n)

                @pl.when(j < n_chunks - 1)
                def _():
                    fire(j + 1, b3n, b2n)

                wait_in(b3, b2)
                compute(b3, b2)
                store(j, b3)
            return carry

        lax.fori_loop(0, (n_chunks - 2) // 6, outer, 0)
        drain_store((n_chunks - 2) % 3)
        drain_store((n_chunks - 1) % 3)

    return fused


def kernel(input_ids, word_emb, pos_emb, pos_emb2, ln_weight, ln_bias):
    b, s = input_ids.shape
    hidden = word_emb.shape[1]
    n = b * s
    ids = input_ids.reshape(n).astype(jnp.int32)
    embeddings, inputs_embeds = _make_sc_fused(n, hidden, s)(
        word_emb, ids, pos_emb[:s], pos_emb2[:s], ln_weight, ln_bias)
    return (embeddings.reshape(b, s, hidden),
            inputs_embeds.reshape(b, s, hidden))


# speedup vs baseline: 1.7290x; 1.0284x over previous
import functools

import jax
import jax.numpy as jnp
from jax import lax
from jax.experimental import pallas as pl
from jax.experimental.pallas import tpu as pltpu
from jax.experimental.pallas import tpu_sc as plsc

EPS = 1e-12

_NUM_CORES = 2
_NUM_SUBCORES = 16
_NW = _NUM_CORES * _NUM_SUBCORES
_FK = 16


@functools.lru_cache(maxsize=None)
def _make_sc_fused(n_tokens: int, hidden: int, s_len: int):
    assert n_tokens % (_NW * _FK) == 0
    per_w = n_tokens // _NW
    n_chunks = per_w // _FK
    n_sl = hidden // 16
    mesh = plsc.VectorSubcoreMesh(core_axis_name="c", subcore_axis_name="s")
    row_t = pltpu.VMEM((_FK, hidden), jnp.float32)

    @functools.partial(
        pl.kernel,
        out_type=(
            jax.ShapeDtypeStruct((n_tokens, hidden), jnp.float32),
            jax.ShapeDtypeStruct((n_tokens, hidden), jnp.float32),
        ),
        mesh=mesh,
        scratch_types=(
            [pltpu.VMEM((per_w,), jnp.int32)]
            + [row_t] * 3
            + [row_t] * 3
            + [row_t] * 2 + [row_t] * 2
            + [pltpu.VMEM((hidden,), jnp.float32)] * 2
            + [pltpu.SemaphoreType.DMA] * 13
        ),
    )
    def fused(word, ids, pos, pos2, w, b, emb_out, ie_out, idx_v,
              r0, r1, r2, e0, e1, e2, p0, p1, q0, q1, wv, bv,
              g0, g1, g2, sp0, sp1, sq0, sq1, sr0, sr1, sr2, se0, se1, se2):
        rows = (r0, r1, r2)
        ebuf = (e0, e1, e2)
        posb = (p0, p1)
        pos2b = (q0, q1)
        gsem = (g0, g1, g2)
        psem = (sp0, sp1)
        qsem = (sq0, sq1)
        rsem = (sr0, sr1, sr2)
        esem = (se0, se1, se2)
        c = lax.axis_index("c")
        s = lax.axis_index("s")
        wid = s * _NUM_CORES + c
        base = wid * per_w
        pos_base = lax.rem(base, s_len)
        pltpu.sync_copy(ids.at[pl.ds(base, per_w)], idx_v)
        pltpu.sync_copy(w, wv)
        pltpu.sync_copy(b, bv)

        def fire(j, b3, b2):
            pltpu.async_copy(
                word.at[idx_v.at[pl.ds(j * _FK, _FK)]], rows[b3], gsem[b3])
            pltpu.async_copy(
                pos.at[pl.ds(pos_base + j * _FK, _FK)], posb[b2], psem[b2])
            pltpu.async_copy(
                pos2.at[pl.ds(pos_base + j * _FK, _FK)], pos2b[b2], qsem[b2])

        def wait_in(b3, b2):
            pltpu.make_async_copy(word.at[pl.ds(0, _FK)], rows[b3], gsem[b3]).wait()
            pltpu.make_async_copy(pos.at[pl.ds(0, _FK)], posb[b2], psem[b2]).wait()
            pltpu.make_async_copy(pos2.at[pl.ds(0, _FK)], pos2b[b2], qsem[b2]).wait()

        def store(j, b3):
            pltpu.async_copy(
                ebuf[b3], emb_out.at[pl.ds(base + j * _FK, _FK)], esem[b3])
            pltpu.async_copy(
                rows[b3], ie_out.at[pl.ds(base + j * _FK, _FK)], rsem[b3])

        def drain_store(b3):
            pltpu.make_async_copy(
                ebuf[b3], emb_out.at[pl.ds(0, _FK)], esem[b3]).wait()
            pltpu.make_async_copy(
                rows[b3], ie_out.at[pl.ds(0, _FK)], rsem[b3]).wait()

        inv_h = 1.0 / hidden
        zero = jnp.zeros((16,), jnp.float32)
        lanes = lax.iota(jnp.int32, 16)

        def compute(b3, b2):
            rb, eb, pb, qb = rows[b3], ebuf[b3], posb[b2], pos2b[b2]

            half = _FK // 2

            def make_sl_body(t0):
                def sl_body(sl, carry):
                    d = pl.ds(sl * 16, 16)
                    ns, nq = [], []
                    for t in range(t0, t0 + half):
                        x = pb[t, d] * rb[t, d] + qb[t, d]
                        eb[t, d] = x
                        ns.append(carry[t - t0] + x)
                        nq.append(carry[half + t - t0] + x * x)
                    return tuple(ns) + tuple(nq)
                return sl_body

            accs_lo = lax.fori_loop(0, n_sl, make_sl_body(0), (zero,) * _FK)
            accs_hi = lax.fori_loop(0, n_sl, make_sl_body(half), (zero,) * _FK)
            accs = (accs_lo[:half] + accs_hi[:half]
                    + accs_lo[half:] + accs_hi[half:])

            def _allreduce(x):
                for sh in (8, 4, 2, 1):
                    perm = lax.bitwise_xor(lanes, jnp.full((16,), sh, jnp.int32))
                    x = x + x.at[perm].get(mode="promise_in_bounds")
                return x

            mean_p = zero
            msq_p = zero
            for t in range(_FK):
                oh = jnp.where(lanes == t, inv_h, 0.0).astype(jnp.float32)
                mean_p = mean_p + _allreduce(accs[t]) * oh
                msq_p = msq_p + _allreduce(accs[_FK + t]) * oh
            v = msq_p - mean_p * mean_p + EPS
            i = lax.bitcast_convert_type(v, jnp.int32)
            i = jnp.full((16,), 0x5F3759DF, jnp.int32) - (i >> 1)
            y = lax.bitcast_convert_type(i, jnp.float32)
            for _ in range(3):
                y = y * (1.5 - 0.5 * v * y * y)
            bms = [mean_p.at[jnp.full((16,), t, jnp.int32)]
                   .get(mode="promise_in_bounds") for t in range(_FK)]
            bys = [y.at[jnp.full((16,), t, jnp.int32)]
                   .get(mode="promise_in_bounds") for t in range(_FK)]

            def make_sl2_body(t0):
                def sl2_body(sl, carry):
                    d = pl.ds(sl * 16, 16)
                    wd = wv[d]
                    bd = bv[d]
                    for t in range(t0, t0 + half):
                        xx = eb[t, d]
                        eb[t, d] = (xx - bms[t]) * bys[t] * wd + bd
                    return carry
                return sl2_body

            lax.fori_loop(0, n_sl, make_sl2_body(0), 0)
            lax.fori_loop(0, n_sl, make_sl2_body(half), 0)

        fire(0, 0, 0)
        fire(1, 1, 1)
        wait_in(0, 0)
        compute(0, 0)
        store(0, 0)
        fire(2, 2, 0)
        wait_in(1, 1)
        compute(1, 1)
        store(1, 1)

        def outer(it, carry):
            g = 2 + 6 * it
            for k in range(6):
                j = g + k
                b3 = (2 + k) % 3
                b2 = (2 + k) % 2
                b3n = (b3 + 1) % 3
                b2n = (b2 + 1) % 2
                drain_store(b3n)

                @pl.when(j < n_chunks - 1)
                def _():
                    fire(j + 1, b3n, b2n)

                wait_in(b3, b2)
                compute(b3, b2)
                store(j, b3)
            return carry

        lax.fori_loop(0, (n_chunks - 2) // 6, outer, 0)
        drain_store((n_chunks - 2) % 3)
        drain_store((n_chunks - 1) % 3)

    return fused


def kernel(input_ids, word_emb, pos_emb, pos_emb2, ln_weight, ln_bias):
    b, s = input_ids.shape
    hidden = word_emb.shape[1]
    n = b * s
    ids = input_ids.reshape(n).astype(jnp.int32)
    embeddings, inputs_embeds = _make_sc_fused(n, hidden, s)(
        word_emb, ids, pos_emb[:s], pos_emb2[:s], ln_weight, ln_bias)
    return (embeddings.reshape(b, s, hidden),
            inputs_embeds.reshape(b, s, hidden))


# speedup vs baseline: 2.5140x; 1.4540x over previous
import functools

import jax
import jax.numpy as jnp
from jax import lax
from jax.experimental import pallas as pl
from jax.experimental.pallas import tpu as pltpu
from jax.experimental.pallas import tpu_sc as plsc

EPS = 1e-12

_NUM_CORES = 2
_NUM_SUBCORES = 16
_NW = _NUM_CORES * _NUM_SUBCORES
_CHUNK = 32
_NBUF = 4


@functools.lru_cache(maxsize=None)
def _make_sc_gather(n_tokens: int, hidden: int):
    assert n_tokens % (_NW * _CHUNK) == 0
    per_w = n_tokens // _NW
    n_chunks = per_w // _CHUNK
    mesh = plsc.VectorSubcoreMesh(core_axis_name="c", subcore_axis_name="s")

    @functools.partial(
        pl.kernel,
        out_type=jax.ShapeDtypeStruct((n_tokens, hidden), jnp.float32),
        mesh=mesh,
        scratch_types=(
            [pltpu.VMEM((per_w,), jnp.int32)]
            + [pltpu.VMEM((_CHUNK, hidden), jnp.float32)] * _NBUF
            + [pltpu.SemaphoreType.DMA] * _NBUF
            + [pltpu.SemaphoreType.DMA] * _NBUF
        ),
    )
    def gather(word_hbm, ids_hbm, out_hbm, idx_v, *bufs_sems):
        bufs = bufs_sems[:_NBUF]
        gsems = bufs_sems[_NBUF:2 * _NBUF]
        ssems = bufs_sems[2 * _NBUF:]
        c = lax.axis_index("c")
        s = lax.axis_index("s")
        wid = s * _NUM_CORES + c
        base = wid * per_w
        pltpu.sync_copy(ids_hbm.at[pl.ds(base, per_w)], idx_v)
        gathers = [None] * n_chunks
        stores = [None] * n_chunks
        store_waited = [False] * n_chunks
        depth = min(_NBUF - 1, n_chunks)
        for j in range(depth):
            gathers[j] = pltpu.async_copy(
                word_hbm.at[idx_v.at[pl.ds(j * _CHUNK, _CHUNK)]],
                bufs[j % _NBUF], gsems[j % _NBUF])
        for j in range(n_chunks):
            gathers[j].wait()
            stores[j] = pltpu.async_copy(
                bufs[j % _NBUF], out_hbm.at[pl.ds(base + j * _CHUNK, _CHUNK)],
                ssems[j % _NBUF])
            nxt = j + depth
            if nxt < n_chunks:
                prev = nxt - _NBUF
                if prev >= 0:
                    stores[prev].wait()
                    store_waited[prev] = True
                gathers[nxt] = pltpu.async_copy(
                    word_hbm.at[idx_v.at[pl.ds(nxt * _CHUNK, _CHUNK)]],
                    bufs[nxt % _NBUF], gsems[nxt % _NBUF])
        for j in range(n_chunks):
            if not store_waited[j]:
                stores[j].wait()

    return gather


def _ln_body(emb_ref, pos_ref, pos2_ref, w_ref, b_ref, out_ref):
    x = pos_ref[...] * emb_ref[...] + pos2_ref[...]
    mean = jnp.mean(x, axis=-1, keepdims=True)
    xc = x - mean
    var = jnp.mean(xc * xc, axis=-1, keepdims=True)
    y = xc * lax.rsqrt(var + EPS)
    out_ref[...] = y * w_ref[...] + b_ref[...]


def _ln_call(emb, pos, pos2, w, b, block_tokens: int):
    n, hidden = emb.shape
    s_len = pos.shape[0]
    assert n % block_tokens == 0 and s_len % block_tokens == 0
    s_blocks = s_len // block_tokens
    batch = n // s_len
    return pl.pallas_call(
        _ln_body,
        grid=(s_blocks, batch),
        in_specs=[
            pl.BlockSpec((block_tokens, hidden), lambda j, bi: (bi * s_blocks + j, 0)),
            pl.BlockSpec((block_tokens, hidden), lambda j, bi: (j, 0)),
            pl.BlockSpec((block_tokens, hidden), lambda j, bi: (j, 0)),
            pl.BlockSpec((1, hidden), lambda j, bi: (0, 0)),
            pl.BlockSpec((1, hidden), lambda j, bi: (0, 0)),
        ],
        out_specs=pl.BlockSpec(
            (block_tokens, hidden), lambda j, bi: (bi * s_blocks + j, 0)),
        out_shape=jax.ShapeDtypeStruct((n, hidden), jnp.float32),
    )(emb, pos, pos2, w, b)


def kernel(input_ids, word_emb, pos_emb, pos_emb2, ln_weight, ln_bias):
    b, s = input_ids.shape
    hidden = word_emb.shape[1]
    n = b * s
    ids = input_ids.reshape(n).astype(jnp.int32)
    inputs_embeds = _make_sc_gather(n, hidden)(word_emb, ids)
    pos = pos_emb[:s]
    pos2 = pos_emb2[:s]
    embeddings = _ln_call(
        inputs_embeds, pos, pos2,
        ln_weight.reshape(1, hidden), ln_bias.reshape(1, hidden),
        block_tokens=2048)
    return (embeddings.reshape(b, s, hidden),
            inputs_embeds.reshape(b, s, hidden))


# speedup vs baseline: 2.5193x; 1.0021x over previous
import functools

import jax
import jax.numpy as jnp
from jax import lax
from jax.experimental import pallas as pl
from jax.experimental.pallas import tpu as pltpu
from jax.experimental.pallas import tpu_sc as plsc

EPS = 1e-12

_NUM_CORES = 2
_NUM_SUBCORES = 16
_NW = _NUM_CORES * _NUM_SUBCORES
_CHUNK = 32
_NBUF = 5


@functools.lru_cache(maxsize=None)
def _make_sc_gather(n_tokens: int, hidden: int):
    assert n_tokens % (_NW * _CHUNK) == 0
    per_w = n_tokens // _NW
    n_chunks = per_w // _CHUNK
    mesh = plsc.VectorSubcoreMesh(core_axis_name="c", subcore_axis_name="s")

    @functools.partial(
        pl.kernel,
        out_type=jax.ShapeDtypeStruct((n_tokens, hidden), jnp.float32),
        mesh=mesh,
        scratch_types=(
            [pltpu.VMEM((per_w,), jnp.int32)]
            + [pltpu.VMEM((_CHUNK, hidden), jnp.float32)] * _NBUF
            + [pltpu.SemaphoreType.DMA] * _NBUF
            + [pltpu.SemaphoreType.DMA] * _NBUF
        ),
    )
    def gather(word_hbm, ids_hbm, out_hbm, idx_v, *bufs_sems):
        bufs = bufs_sems[:_NBUF]
        gsems = bufs_sems[_NBUF:2 * _NBUF]
        ssems = bufs_sems[2 * _NBUF:]
        c = lax.axis_index("c")
        s = lax.axis_index("s")
        wid = s * _NUM_CORES + c
        base = wid * per_w
        pltpu.sync_copy(ids_hbm.at[pl.ds(base, per_w)], idx_v)
        gathers = [None] * n_chunks
        stores = [None] * n_chunks
        store_waited = [False] * n_chunks
        depth = min(_NBUF - 1, n_chunks)
        for j in range(depth):
            gathers[j] = pltpu.async_copy(
                word_hbm.at[idx_v.at[pl.ds(j * _CHUNK, _CHUNK)]],
                bufs[j % _NBUF], gsems[j % _NBUF])
        for j in range(n_chunks):
            gathers[j].wait()
            stores[j] = pltpu.async_copy(
                bufs[j % _NBUF], out_hbm.at[pl.ds(base + j * _CHUNK, _CHUNK)],
                ssems[j % _NBUF])
            nxt = j + depth
            if nxt < n_chunks:
                prev = nxt - _NBUF
                if prev >= 0:
                    stores[prev].wait()
                    store_waited[prev] = True
                gathers[nxt] = pltpu.async_copy(
                    word_hbm.at[idx_v.at[pl.ds(nxt * _CHUNK, _CHUNK)]],
                    bufs[nxt % _NBUF], gsems[nxt % _NBUF])
        for j in range(n_chunks):
            if not store_waited[j]:
                stores[j].wait()

    return gather


def _ln_body(emb_ref, pos_ref, pos2_ref, w_ref, b_ref, out_ref):
    x = pos_ref[...] * emb_ref[...] + pos2_ref[...]
    mean = jnp.mean(x, axis=-1, keepdims=True)
    xc = x - mean
    var = jnp.mean(xc * xc, axis=-1, keepdims=True)
    y = xc * lax.rsqrt(var + EPS)
    out_ref[...] = y * w_ref[...] + b_ref[...]


def _ln_call(emb, pos, pos2, w, b, block_tokens: int):
    n, hidden = emb.shape
    s_len = pos.shape[0]
    assert n % block_tokens == 0 and s_len % block_tokens == 0
    s_blocks = s_len // block_tokens
    batch = n // s_len
    return pl.pallas_call(
        _ln_body,
        grid=(s_blocks, batch),
        in_specs=[
            pl.BlockSpec((block_tokens, hidden), lambda j, bi: (bi * s_blocks + j, 0)),
            pl.BlockSpec((block_tokens, hidden), lambda j, bi: (j, 0)),
            pl.BlockSpec((block_tokens, hidden), lambda j, bi: (j, 0)),
            pl.BlockSpec((1, hidden), lambda j, bi: (0, 0)),
            pl.BlockSpec((1, hidden), lambda j, bi: (0, 0)),
        ],
        out_specs=pl.BlockSpec(
            (block_tokens, hidden), lambda j, bi: (bi * s_blocks + j, 0)),
        out_shape=jax.ShapeDtypeStruct((n, hidden), jnp.float32),
    )(emb, pos, pos2, w, b)


def kernel(input_ids, word_emb, pos_emb, pos_emb2, ln_weight, ln_bias):
    b, s = input_ids.shape
    hidden = word_emb.shape[1]
    n = b * s
    ids = input_ids.reshape(n).astype(jnp.int32)
    inputs_embeds = _make_sc_gather(n, hidden)(word_emb, ids)
    pos = pos_emb[:s]
    pos2 = pos_emb2[:s]
    embeddings = _ln_call(
        inputs_embeds, pos, pos2,
        ln_weight.reshape(1, hidden), ln_bias.reshape(1, hidden),
        block_tokens=2048)
    return (embeddings.reshape(b, s, hidden),
            inputs_embeds.reshape(b, s, hidden))
